# Initial kernel scaffold; baseline (speedup 1.0000x reference)
import jax
import jax.numpy as jnp
from jax.experimental import pallas as pl


def kernel(x_num, num_col_input_ids, num_att_mask, x_cat_input_ids, cat_att_mask, x_bin_input_ids, bin_att_mask, W_emb, ln_gamma, ln_beta, num_bias, W_align):
    raise NotImplementedError("write your pallas kernel here")


# speedup vs baseline: 3.4235x; 3.4235x over previous
import functools

import jax
import jax.numpy as jnp
from jax import lax
from jax.experimental import pallas as pl
from jax.experimental.pallas import tpu as pltpu
from jax.experimental.pallas import tpu_sc as plsc

NC, NS, L = 2, 16, 16
NW = NC * NS

_ROWS_PER_BLOCK = 800


def _table_body(w_ref, g_ref, b_ref, wa_ref, nb_ref, at_ref, c2_ref):
    e = w_ref[:]
    mu = jnp.mean(e, axis=1, keepdims=True)
    xc = e - mu
    var = jnp.mean(xc * xc, axis=1, keepdims=True)
    y = xc * lax.rsqrt(var + 1e-5) * g_ref[:] + b_ref[:]
    at_ref[:] = lax.dot_general(y, wa_ref[:], (((1,), (1,)), ((), ())),
                                preferred_element_type=jnp.float32)

    @pl.when(pl.program_id(0) == 0)
    def _():
        c2_ref[:] = lax.dot_general(nb_ref[:], wa_ref[:],
                                    (((1,), (1,)), ((), ())),
                                    preferred_element_type=jnp.float32)


def _aligned_table(W_emb, gamma2d, beta2d, W_align, nb2d):
    V, D = W_emb.shape
    nblk = V // _ROWS_PER_BLOCK
    return pl.pallas_call(
        _table_body,
        grid=(nblk,),
        in_specs=[
            pl.BlockSpec((_ROWS_PER_BLOCK, D), lambda i: (i, 0)),
            pl.BlockSpec((1, D), lambda i: (0, 0)),
            pl.BlockSpec((1, D), lambda i: (0, 0)),
            pl.BlockSpec((D, D), lambda i: (0, 0)),
            pl.BlockSpec((1, D), lambda i: (0, 0)),
        ],
        out_specs=[
            pl.BlockSpec((_ROWS_PER_BLOCK, D), lambda i: (i, 0)),
            pl.BlockSpec((1, D), lambda i: (0, 0)),
        ],
        out_shape=[
            jax.ShapeDtypeStruct((V, D), jnp.float32),
            jax.ShapeDtypeStruct((1, D), jnp.float32),
        ],
    )(W_emb, gamma2d, beta2d, W_align, nb2d)


def _lane_bcast(v, lane):
    idx = jnp.full((L,), lane, jnp.int32)
    return v.at[idx].get(mode="promise_in_bounds", unique_indices=False)


def _make_sc_assemble(B, D, n_num, num_tok, n_gath, n_out):
    assert B % NW == 0
    bpw = B // NW
    nc = D // L
    npad = ((n_num + L - 1) // L) * L
    mesh = plsc.VectorSubcoreMesh(core_axis_name="c", subcore_axis_name="s")

    @functools.partial(
        pl.kernel,
        mesh=mesh,
        out_type=jax.ShapeDtypeStruct((B, n_out, D), jnp.float32),
        scratch_types=[
            pltpu.VMEM((bpw, n_gath), jnp.int32),
            pltpu.VMEM((bpw, npad), jnp.float32),
            pltpu.VMEM((n_num * num_tok,), jnp.int32),
            pltpu.VMEM((n_num * num_tok,), jnp.float32),
            pltpu.VMEM((n_num * num_tok, D), jnp.float32),
            pltpu.VMEM((n_num, D), jnp.float32),
            pltpu.VMEM((1, D), jnp.float32),
            pltpu.VMEM((n_out, D), jnp.float32),
            pltpu.SemaphoreType.DMA,
        ],
    )
    def sc_kernel(at_hbm, ids_hbm, xnum_hbm, numidx_hbm, nummask_hbm, c2_hbm,
                  out_hbm, ids_v, xnum_v, numidx_v, nummask_v, numrows_v,
                  a_v, c2_v, buf, gsem):
        wid = lax.axis_index("s") * NC + lax.axis_index("c")
        base = wid * bpw

        pltpu.sync_copy(ids_hbm.at[pl.ds(base, bpw)], ids_v)
        pltpu.sync_copy(xnum_hbm.at[pl.ds(base, bpw)], xnum_v)
        pltpu.sync_copy(numidx_hbm, numidx_v)
        pltpu.sync_copy(nummask_hbm, nummask_v)
        pltpu.sync_copy(c2_hbm, c2_v)

        pltpu.async_copy(at_hbm.at[numidx_v], numrows_v, gsem).wait()

        def a_body(i, _):
            accs = [jnp.zeros((L,), jnp.float32) for _ in range(nc)]
            den = jnp.zeros((L,), jnp.float32)
            for t in range(num_tok):
                flat = i * num_tok + t
                c0 = (flat // L) * L
                mvec = nummask_v[pl.ds(c0, L)]
                m = _lane_bcast(mvec, flat - c0)
                den = den + m
                for c in range(nc):
                    accs[c] = accs[c] + m * numrows_v[flat, pl.ds(c * L, L)]
            for c in range(nc):
                a_v[i, pl.ds(c * L, L)] = accs[c] / den
            return 0

        lax.fori_loop(0, n_num, a_body, 0)

        def b_body(bl, _):
            pltpu.async_copy(at_hbm.at[ids_v.at[bl]],
                             buf.at[pl.ds(n_num, n_gath)], gsem).wait()

            def row_body(i, _):
                c0 = (i // L) * L
                bi = _lane_bcast(xnum_v[bl, pl.ds(c0, L)], i - c0)
                for c in range(nc):
                    buf[i, pl.ds(c * L, L)] = (
                        bi * a_v[i, pl.ds(c * L, L)] + c2_v[0, pl.ds(c * L, L)])
                return 0

            lax.fori_loop(0, n_num, row_body, 0)
            pltpu.sync_copy(buf, out_hbm.at[base + bl])
            return 0

        lax.fori_loop(0, bpw, b_body, 0)

    return sc_kernel


def kernel(x_num, num_col_input_ids, num_att_mask, x_cat_input_ids,
           cat_att_mask, x_bin_input_ids, bin_att_mask, W_emb, ln_gamma,
           ln_beta, num_bias, W_align):
    V, D = W_emb.shape
    B, n_num = x_num.shape
    num_tok = num_col_input_ids.shape[1]
    cat_len = x_cat_input_ids.shape[1]
    bin_len = x_bin_input_ids.shape[1]
    n_gath = cat_len + bin_len
    n_out = n_num + n_gath

    AT, c2 = _aligned_table(
        W_emb, ln_gamma.reshape(1, D), ln_beta.reshape(1, D), W_align,
        num_bias.reshape(1, D))

    ids_all = jnp.concatenate(
        [x_cat_input_ids, x_bin_input_ids], axis=1).astype(jnp.int32)
    numidx = num_col_input_ids.reshape(-1).astype(jnp.int32)
    nummask = num_att_mask.reshape(-1).astype(jnp.float32)

    npad = ((n_num + 15) // 16) * 16
    xnum_p = jnp.pad(x_num, ((0, 0), (0, npad - n_num)))

    sc = _make_sc_assemble(B, D, n_num, num_tok, n_gath, n_out)
    embedding = sc(AT, ids_all, xnum_p, numidx, nummask, c2)

    attention_mask = jnp.concatenate(
        [jnp.ones((B, n_num), jnp.float32), cat_att_mask, bin_att_mask],
        axis=1)
    return embedding, attention_mask
